# Initial kernel scaffold; baseline (speedup 1.0000x reference)
import jax
import jax.numpy as jnp
from jax.experimental import pallas as pl


def kernel(node, edge, edge_index, graph_idx, Wn1, bn1, Wn2, bn2, We1, be1, We2, be2, Wm1, bm1, Wm2, bm2, Wu1, bu1, Wu2, bu2, Wa1, ba1, Wa2, ba2):
    raise NotImplementedError("write your pallas kernel here")


# speedup vs baseline: 2.3567x; 2.3567x over previous
import functools

import jax
import jax.numpy as jnp
from jax import lax
from jax.experimental import pallas as pl
from jax.experimental.pallas import tpu as pltpu
from jax.experimental.pallas import tpu_sc as plsc

F32 = jnp.float32
G = 64
_SC_STAGE = 2


def _node_enc_body(node_ref, w1_ref, b1_ref, w2_ref, b2_ref, h_ref, hp_ref):
    x = node_ref[...]
    t = jnp.maximum(x * w1_ref[...] + b1_ref[...], 0.0)
    h = jnp.dot(t, w2_ref[...], preferred_element_type=F32) + b2_ref[...]
    h_ref[...] = h
    hp_ref[...] = jnp.concatenate([h, jnp.zeros_like(h)], axis=1)


def _edge_enc_body(edge_ref, w1_ref, b1_ref, w2_ref, b2_ref, ef_ref):
    x = edge_ref[...]
    t = jnp.maximum(x * w1_ref[...] + b1_ref[...], 0.0)
    ef_ref[...] = jnp.dot(t, w2_ref[...], preferred_element_type=F32) + b2_ref[...]


def _msg_body(hid, hf_ref, ht_ref, ef_ref, wm1_ref, bm1_ref, wm2_ref, bm2_ref,
              msg_ref):
    m = jnp.concatenate([hf_ref[...][:, :hid], ht_ref[...][:, :hid],
                         ef_ref[...]], axis=1)
    t = jnp.maximum(jnp.dot(m, wm1_ref[...], preferred_element_type=F32)
                    + bm1_ref[...], 0.0)
    msg = jnp.dot(t, wm2_ref[...], preferred_element_type=F32) + bm2_ref[...]
    msg_ref[...] = jnp.concatenate([msg, jnp.zeros_like(msg)], axis=1)


def _update_body(nc, n, hid, h_ref, s_ref, wu1_ref, bu1_ref, wu2_ref, bu2_ref,
                 h2_ref, hp_ref):
    agg = s_ref[0][:n, :hid]
    for c in range(1, nc):
        agg = agg + s_ref[c][:n, :hid]
    u = jnp.concatenate([h_ref[...], agg], axis=1)
    t = jnp.maximum(jnp.dot(u, wu1_ref[...], preferred_element_type=F32)
                    + bu1_ref[...], 0.0)
    h2 = jnp.dot(t, wu2_ref[...], preferred_element_type=F32) + bu2_ref[...]
    h2_ref[...] = h2
    hp_ref[...] = jnp.concatenate([h2, jnp.zeros_like(h2)], axis=1)


def _final_body(nc, n, hid, gr, h_ref, s_ref, wu1_ref, bu1_ref, wu2_ref, bu2_ref,
                wa1_ref, ba1_ref, wa2_ref, ba2_ref, gi_ref, out_ref):
    agg = s_ref[0][:n, :hid]
    for c in range(1, nc):
        agg = agg + s_ref[c][:n, :hid]
    u = jnp.concatenate([h_ref[...], agg], axis=1)
    t = jnp.maximum(jnp.dot(u, wu1_ref[...], preferred_element_type=F32)
                    + bu1_ref[...], 0.0)
    h2 = jnp.dot(t, wu2_ref[...], preferred_element_type=F32) + bu2_ref[...]
    ga = jnp.maximum(jnp.dot(h2, wa1_ref[...], preferred_element_type=F32)
                     + ba1_ref[...], 0.0)
    g = jnp.dot(ga, wa2_ref[...], preferred_element_type=F32) + ba2_ref[...]
    gated = jax.nn.sigmoid(g[:, :gr]) * g[:, gr:]
    onehot = (gi_ref[...] == lax.broadcasted_iota(jnp.int32, (1, G), 1)).astype(F32)
    out_ref[...] = lax.dot_general(onehot, gated, (((0,), (0,)), ((), ())),
                                   preferred_element_type=F32)


def _sc_info():
    info = plsc.get_sparse_core_info()
    return info.num_cores, info.num_subcores


def _make_gather(n_nodes, n_edges, d):
    nc, ns = _sc_info()
    nw = nc * ns
    ch = 128
    assert n_edges % nw == 0
    epw = n_edges // nw
    assert epw % 8 == 0
    nch = epw // ch
    tail = epw - nch * ch
    mesh = plsc.VectorSubcoreMesh(core_axis_name="c", subcore_axis_name="s")

    scratch = [pltpu.VMEM((ch,), jnp.int32), pltpu.VMEM((ch,), jnp.int32),
               pltpu.VMEM((ch, d), F32), pltpu.VMEM((ch, d), F32),
               pltpu.SemaphoreType.DMA]
    if tail:
        scratch += [pltpu.VMEM((tail,), jnp.int32),
                    pltpu.VMEM((tail,), jnp.int32),
                    pltpu.VMEM((tail, d), F32), pltpu.VMEM((tail, d), F32)]

    @functools.partial(
        pl.kernel, mesh=mesh,
        out_type=[jax.ShapeDtypeStruct((n_edges, d), F32),
                  jax.ShapeDtypeStruct((n_edges, d), F32)],
        scratch_types=scratch)
    def k(hp_hbm, frm_hbm, to_hbm, hf_out, ht_out, *scr):
        if tail:
            (idx_f, idx_t, rows_f, rows_t, sem,
             idx_f2, idx_t2, rows_f2, rows_t2) = scr
        else:
            idx_f, idx_t, rows_f, rows_t, sem = scr
        wid = lax.axis_index("s") * nc + lax.axis_index("c")
        base0 = wid * epw

        def do_chunk(base, cnt, ixf, ixt, rf, rt):
            pltpu.sync_copy(frm_hbm.at[pl.ds(base, cnt)], ixf)
            pltpu.sync_copy(to_hbm.at[pl.ds(base, cnt)], ixt)
            cf = pltpu.async_copy(hp_hbm.at[ixf], rf, sem)
            ct = pltpu.async_copy(hp_hbm.at[ixt], rt, sem)
            cf.wait()
            ct.wait()
            pltpu.sync_copy(rf, hf_out.at[pl.ds(base, cnt)])
            pltpu.sync_copy(rt, ht_out.at[pl.ds(base, cnt)])

        def body(kk, _):
            do_chunk(pl.multiple_of(base0 + kk * ch, 8), ch,
                     idx_f, idx_t, rows_f, rows_t)
            return 0
        lax.fori_loop(0, nch, body, 0)
        if tail:
            do_chunk(pl.multiple_of(base0 + nch * ch, 8), tail,
                     idx_f2, idx_t2, rows_f2, rows_t2)

    return k


def _make_scatter(n_nodes, n_edges, hid):
    nc, ns = _sc_info()
    nw = nc * ns
    ch = 128
    epw = n_edges // nw
    nch = epw // ch
    tail = epw - nch * ch
    n_pad = -(-n_nodes // ch) * ch
    nrch = n_pad // ch
    kper = -(-nrch // ns)
    mesh = plsc.VectorSubcoreMesh(core_axis_name="c", subcore_axis_name="s")

    w = 2 * hid
    scratch = [pltpu.VMEM((ch,), jnp.int32), pltpu.VMEM((ch, w), F32),
               pltpu.VMEM_SHARED((n_pad, w), F32), pltpu.SemaphoreType.DMA]
    if tail:
        scratch += [pltpu.VMEM((tail,), jnp.int32),
                    pltpu.VMEM((tail, w), F32)]

    @functools.partial(
        pl.kernel, mesh=mesh,
        out_type=jax.ShapeDtypeStruct((nc, n_pad, w), F32),
        scratch_types=scratch)
    def k(msg_hbm, to_hbm, s_out, *scr):
        if tail:
            idx_t, rows, acc, sem, idx_t2, rows2 = scr
        else:
            idx_t, rows, acc, sem = scr
        cid = lax.axis_index("c")
        sid = lax.axis_index("s")
        wid = sid * nc + cid
        base0 = wid * epw

        def zrow(r, _):
            for c in range(w // 16):
                rows[r, pl.ds(c * 16, 16)] = jnp.zeros((16,), F32)
            return 0
        lax.fori_loop(0, ch, zrow, 0)

        def zchunk(kk, _):
            c = lax.min(sid + kk * ns, nrch - 1)
            pltpu.sync_copy(rows, acc.at[pl.ds(pl.multiple_of(c * ch, 8), ch)])
            return 0
        lax.fori_loop(0, kper, zchunk, 0)
        plsc.subcore_barrier()

        def do_chunk(base, cnt, ixt, rr):
            pltpu.sync_copy(to_hbm.at[pl.ds(base, cnt)], ixt)
            pltpu.sync_copy(msg_hbm.at[pl.ds(base, cnt)], rr)
            pltpu.sync_copy(rr, acc.at[ixt], add=True)

        def body(kk, _):
            do_chunk(pl.multiple_of(base0 + kk * ch, 8), ch, idx_t, rows)
            return 0
        lax.fori_loop(0, nch, body, 0)
        if tail:
            do_chunk(pl.multiple_of(base0 + nch * ch, 8), tail, idx_t2, rows2)
        plsc.subcore_barrier()

        def ochunk(kk, _):
            c = lax.min(sid + kk * ns, nrch - 1)
            sl = pl.ds(pl.multiple_of(c * ch, 8), ch)
            pltpu.sync_copy(acc.at[sl], rows)
            pltpu.sync_copy(rows, s_out.at[cid, sl])
            return 0
        lax.fori_loop(0, kper, ochunk, 0)

    return k


def kernel(node, edge, edge_index, graph_idx, Wn1, bn1, Wn2, bn2, We1, be1,
           We2, be2, Wm1, bm1, Wm2, bm2, Wu1, bu1, Wu2, bu2, Wa1, ba1,
           Wa2, ba2):
    n = node.shape[0]
    e = edge.shape[0]
    nlayers = Wm1.shape[0]
    enc = Wn2.shape[1]
    hid = Wm2.shape[1]
    gr = Wa2.shape[1] // 2
    frm = edge_index[0]
    to = edge_index[1]
    row2 = lambda v: v.reshape(1, -1)

    h, hp = pl.pallas_call(
        _node_enc_body,
        out_shape=[jax.ShapeDtypeStruct((n, enc), F32),
                   jax.ShapeDtypeStruct((n, 2 * enc), F32)],
    )(node, Wn1, row2(bn1), Wn2, row2(bn2))

    be_blk = 2000
    assert e % be_blk == 0
    full = lambda shape: pl.BlockSpec(shape, lambda i: (0,) * len(shape))
    ef = pl.pallas_call(
        _edge_enc_body,
        grid=(e // be_blk,),
        in_specs=[pl.BlockSpec((be_blk, 1), lambda i: (i, 0)),
                  full(We1.shape), full((1, We1.shape[1])),
                  full(We2.shape), full((1, We2.shape[1]))],
        out_specs=pl.BlockSpec((be_blk, enc), lambda i: (i, 0)),
        out_shape=jax.ShapeDtypeStruct((e, enc), F32),
    )(edge, We1, row2(be1), We2, row2(be2))

    if _SC_STAGE >= 1:
        sc_gather = _make_gather(n, e, 2 * enc)
    if _SC_STAGE >= 2:
        sc_scatter = _make_scatter(n, e, hid)
    ncores, _ = _sc_info()

    for i in range(nlayers):
        if _SC_STAGE >= 1:
            hf, ht = sc_gather(hp, frm, to)
        else:
            hf = hp[frm]
            ht = hp[to]
        msg = pl.pallas_call(
            functools.partial(_msg_body, hid),
            grid=(e // be_blk,),
            in_specs=[pl.BlockSpec((be_blk, 2 * enc), lambda i: (i, 0)),
                      pl.BlockSpec((be_blk, 2 * enc), lambda i: (i, 0)),
                      pl.BlockSpec((be_blk, enc), lambda i: (i, 0)),
                      full(Wm1[0].shape), full((1, hid)),
                      full(Wm2[0].shape), full((1, hid))],
            out_specs=pl.BlockSpec((be_blk, 2 * hid), lambda i: (i, 0)),
            out_shape=jax.ShapeDtypeStruct((e, 2 * hid), F32),
        )(hf, ht, ef, Wm1[i], row2(bm1[i]), Wm2[i], row2(bm2[i]))
        if _SC_STAGE >= 2:
            s = sc_scatter(msg, to)
        else:
            s_flat = jax.ops.segment_sum(msg, to, num_segments=n)
            s = jnp.stack([s_flat, jnp.zeros_like(s_flat)])
        if i + 1 < nlayers:
            h, hp = pl.pallas_call(
                functools.partial(_update_body, ncores, n, hid),
                out_shape=[jax.ShapeDtypeStruct((n, enc), F32),
                           jax.ShapeDtypeStruct((n, 2 * enc), F32)],
            )(h, s, Wu1[i], row2(bu1[i]), Wu2[i], row2(bu2[i]))
        else:
            out = pl.pallas_call(
                functools.partial(_final_body, ncores, n, hid, gr),
                out_shape=jax.ShapeDtypeStruct((G, gr), F32),
            )(h, s, Wu1[i], row2(bu1[i]), Wu2[i], row2(bu2[i]),
              Wa1, row2(ba1), Wa2, row2(ba2), graph_idx.reshape(n, 1))
    return out


# speedup vs baseline: 2.9030x; 1.2318x over previous
import functools

import jax
import jax.numpy as jnp
from jax import lax
from jax.experimental import pallas as pl
from jax.experimental.pallas import tpu as pltpu
from jax.experimental.pallas import tpu_sc as plsc

F32 = jnp.float32
G = 64
_SC_STAGE = 2


def _node_enc_body(node_ref, w1_ref, b1_ref, w2_ref, b2_ref, h_ref, hp_ref):
    x = node_ref[...]
    t = jnp.maximum(x * w1_ref[...] + b1_ref[...], 0.0)
    h = jnp.dot(t, w2_ref[...], preferred_element_type=F32) + b2_ref[...]
    h_ref[...] = h
    hp_ref[...] = jnp.concatenate([h, jnp.zeros_like(h)], axis=1)


def _edge_enc_body(edge_ref, w1_ref, b1_ref, w2_ref, b2_ref, ef_ref):
    x = edge_ref[...]
    t = jnp.maximum(x * w1_ref[...] + b1_ref[...], 0.0)
    ef_ref[...] = jnp.dot(t, w2_ref[...], preferred_element_type=F32) + b2_ref[...]


def _msg_body(hid, hf_ref, ht_ref, ef_ref, wm1_ref, bm1_ref, wm2_ref, bm2_ref,
              msg_ref):
    m = jnp.concatenate([hf_ref[...][:, :hid], ht_ref[...][:, :hid],
                         ef_ref[...]], axis=1)
    t = jnp.maximum(jnp.dot(m, wm1_ref[...], preferred_element_type=F32)
                    + bm1_ref[...], 0.0)
    msg = jnp.dot(t, wm2_ref[...], preferred_element_type=F32) + bm2_ref[...]
    msg_ref[...] = jnp.concatenate([msg, jnp.zeros_like(msg)], axis=1)


def _update_body(nc, n, hid, h_ref, s_ref, wu1_ref, bu1_ref, wu2_ref, bu2_ref,
                 h2_ref, hp_ref):
    agg = s_ref[0][:n, :hid]
    for c in range(1, nc):
        agg = agg + s_ref[c][:n, :hid]
    u = jnp.concatenate([h_ref[...], agg], axis=1)
    t = jnp.maximum(jnp.dot(u, wu1_ref[...], preferred_element_type=F32)
                    + bu1_ref[...], 0.0)
    h2 = jnp.dot(t, wu2_ref[...], preferred_element_type=F32) + bu2_ref[...]
    h2_ref[...] = h2
    hp_ref[...] = jnp.concatenate([h2, jnp.zeros_like(h2)], axis=1)


def _final_body(nc, n, hid, gr, h_ref, s_ref, wu1_ref, bu1_ref, wu2_ref, bu2_ref,
                wa1_ref, ba1_ref, wa2_ref, ba2_ref, gi_ref, out_ref):
    agg = s_ref[0][:n, :hid]
    for c in range(1, nc):
        agg = agg + s_ref[c][:n, :hid]
    u = jnp.concatenate([h_ref[...], agg], axis=1)
    t = jnp.maximum(jnp.dot(u, wu1_ref[...], preferred_element_type=F32)
                    + bu1_ref[...], 0.0)
    h2 = jnp.dot(t, wu2_ref[...], preferred_element_type=F32) + bu2_ref[...]
    ga = jnp.maximum(jnp.dot(h2, wa1_ref[...], preferred_element_type=F32)
                     + ba1_ref[...], 0.0)
    g = jnp.dot(ga, wa2_ref[...], preferred_element_type=F32) + ba2_ref[...]
    gated = jax.nn.sigmoid(g[:, :gr]) * g[:, gr:]
    onehot = (gi_ref[...] == lax.broadcasted_iota(jnp.int32, (1, G), 1)).astype(F32)
    out_ref[...] = lax.dot_general(onehot, gated, (((0,), (0,)), ((), ())),
                                   preferred_element_type=F32)


def _sc_info():
    info = plsc.get_sparse_core_info()
    return info.num_cores, info.num_subcores


def _make_gather(n_nodes, n_edges, d):
    nc, ns = _sc_info()
    nw = nc * ns
    ch = 128
    assert n_edges % nw == 0
    epw = n_edges // nw
    assert epw % 8 == 0
    nch = epw // ch
    tail = epw - nch * ch
    mesh = plsc.VectorSubcoreMesh(core_axis_name="c", subcore_axis_name="s")

    scratch = []
    for _ in range(2):
        scratch += [pltpu.VMEM((ch,), jnp.int32), pltpu.VMEM((ch,), jnp.int32),
                    pltpu.VMEM((ch, d), F32), pltpu.VMEM((ch, d), F32),
                    pltpu.SemaphoreType.DMA, pltpu.SemaphoreType.DMA,
                    pltpu.SemaphoreType.DMA]
    if tail:
        scratch += [pltpu.VMEM((tail,), jnp.int32),
                    pltpu.VMEM((tail,), jnp.int32),
                    pltpu.VMEM((tail, d), F32), pltpu.VMEM((tail, d), F32)]

    @functools.partial(
        pl.kernel, mesh=mesh,
        out_type=[jax.ShapeDtypeStruct((n_edges, d), F32),
                  jax.ShapeDtypeStruct((n_edges, d), F32)],
        scratch_types=scratch)
    def k(hp_hbm, frm_hbm, to_hbm, hf_out, ht_out, *scr):
        slots = [scr[0:7], scr[7:14]]
        wid = lax.axis_index("s") * nc + lax.axis_index("c")
        base0 = wid * epw
        cbase = lambda kk: pl.multiple_of(base0 + kk * ch, 8)

        def start_idx(kk, s):
            ixf, ixt, rf, rt, si, sg, so = slots[s]
            b = cbase(kk)
            return (pltpu.async_copy(frm_hbm.at[pl.ds(b, ch)], ixf, si),
                    pltpu.async_copy(to_hbm.at[pl.ds(b, ch)], ixt, si))

        pend_idx = [None, None]
        pend_out = [None, None]
        if nch > 0:
            pend_idx[0] = start_idx(0, 0)
        for kk in range(nch):
            s = kk & 1
            ixf, ixt, rf, rt, si, sg, so = slots[s]
            for dsc in pend_idx[s]:
                dsc.wait()
            if pend_out[s] is not None:
                for dsc in pend_out[s]:
                    dsc.wait()
                pend_out[s] = None
            ga = pltpu.async_copy(hp_hbm.at[ixf], rf, sg)
            gb = pltpu.async_copy(hp_hbm.at[ixt], rt, sg)
            if kk + 1 < nch:
                pend_idx[s ^ 1] = start_idx(kk + 1, s ^ 1)
            ga.wait()
            gb.wait()
            b = cbase(kk)
            pend_out[s] = (pltpu.async_copy(rf, hf_out.at[pl.ds(b, ch)], so),
                           pltpu.async_copy(rt, ht_out.at[pl.ds(b, ch)], so))
        for s in (0, 1):
            if pend_out[s] is not None:
                for dsc in pend_out[s]:
                    dsc.wait()
        if tail:
            idx_f2, idx_t2, rows_f2, rows_t2 = scr[14:18]
            sem = slots[0][5]
            b = pl.multiple_of(base0 + nch * ch, 8)
            pltpu.sync_copy(frm_hbm.at[pl.ds(b, tail)], idx_f2)
            pltpu.sync_copy(to_hbm.at[pl.ds(b, tail)], idx_t2)
            cf = pltpu.async_copy(hp_hbm.at[idx_f2], rows_f2, sem)
            ct = pltpu.async_copy(hp_hbm.at[idx_t2], rows_t2, sem)
            cf.wait()
            ct.wait()
            pltpu.sync_copy(rows_f2, hf_out.at[pl.ds(b, tail)])
            pltpu.sync_copy(rows_t2, ht_out.at[pl.ds(b, tail)])

    return k


def _make_scatter(n_nodes, n_edges, hid):
    nc, ns = _sc_info()
    nw = nc * ns
    ch = 128
    epw = n_edges // nw
    nch = epw // ch
    tail = epw - nch * ch
    n_pad = -(-n_nodes // ch) * ch
    nrch = n_pad // ch
    kper = -(-nrch // ns)
    mesh = plsc.VectorSubcoreMesh(core_axis_name="c", subcore_axis_name="s")

    w = 2 * hid
    scratch = []
    for _ in range(2):
        scratch += [pltpu.VMEM((ch,), jnp.int32), pltpu.VMEM((ch, w), F32),
                    pltpu.SemaphoreType.DMA, pltpu.SemaphoreType.DMA]
    scratch += [pltpu.VMEM_SHARED((n_pad, w), F32)]
    if tail:
        scratch += [pltpu.VMEM((tail,), jnp.int32),
                    pltpu.VMEM((tail, w), F32)]

    @functools.partial(
        pl.kernel, mesh=mesh,
        out_type=jax.ShapeDtypeStruct((nc, n_pad, w), F32),
        scratch_types=scratch)
    def k(msg_hbm, to_hbm, s_out, *scr):
        slots = [scr[0:4], scr[4:8]]
        acc = scr[8]
        idx_t, rows = slots[0][0], slots[0][1]
        cid = lax.axis_index("c")
        sid = lax.axis_index("s")
        wid = sid * nc + cid
        base0 = wid * epw

        def zrow(r, _):
            for c in range(w // 16):
                rows[r, pl.ds(c * 16, 16)] = jnp.zeros((16,), F32)
            return 0
        lax.fori_loop(0, ch, zrow, 0)

        def zchunk(kk, _):
            c = lax.min(sid + kk * ns, nrch - 1)
            pltpu.sync_copy(rows, acc.at[pl.ds(pl.multiple_of(c * ch, 8), ch)])
            return 0
        lax.fori_loop(0, kper, zchunk, 0)
        plsc.subcore_barrier()

        def start_load(kk, s):
            ixt, rr, si, ss = slots[s]
            b = pl.multiple_of(base0 + kk * ch, 8)
            return (pltpu.async_copy(to_hbm.at[pl.ds(b, ch)], ixt, si),
                    pltpu.async_copy(msg_hbm.at[pl.ds(b, ch)], rr, si))

        pend_load = [None, None]
        pend_sc = [None, None]
        if nch > 0:
            pend_load[0] = start_load(0, 0)
        for kk in range(nch):
            s = kk & 1
            ixt_s, rr_s, si, ss = slots[s]
            for dsc in pend_load[s]:
                dsc.wait()
            sc = pltpu.async_copy(rr_s, acc.at[ixt_s], sem=ss, add=True)
            if kk + 1 < nch:
                if pend_sc[s ^ 1] is not None:
                    pend_sc[s ^ 1].wait()
                pend_load[s ^ 1] = start_load(kk + 1, s ^ 1)
            pend_sc[s] = sc
        for s in (0, 1):
            if pend_sc[s] is not None:
                pend_sc[s].wait()
        if tail:
            idx_t2, rows2 = scr[9], scr[10]
            b = pl.multiple_of(base0 + nch * ch, 8)
            pltpu.sync_copy(to_hbm.at[pl.ds(b, tail)], idx_t2)
            pltpu.sync_copy(msg_hbm.at[pl.ds(b, tail)], rows2)
            pltpu.sync_copy(rows2, acc.at[idx_t2], add=True)
        plsc.subcore_barrier()

        def ochunk(kk, _):
            c = lax.min(sid + kk * ns, nrch - 1)
            sl = pl.ds(pl.multiple_of(c * ch, 8), ch)
            pltpu.sync_copy(acc.at[sl], rows)
            pltpu.sync_copy(rows, s_out.at[cid, sl])
            return 0
        lax.fori_loop(0, kper, ochunk, 0)

    return k


def kernel(node, edge, edge_index, graph_idx, Wn1, bn1, Wn2, bn2, We1, be1,
           We2, be2, Wm1, bm1, Wm2, bm2, Wu1, bu1, Wu2, bu2, Wa1, ba1,
           Wa2, ba2):
    n = node.shape[0]
    e = edge.shape[0]
    nlayers = Wm1.shape[0]
    enc = Wn2.shape[1]
    hid = Wm2.shape[1]
    gr = Wa2.shape[1] // 2
    frm = edge_index[0]
    to = edge_index[1]
    row2 = lambda v: v.reshape(1, -1)

    h, hp = pl.pallas_call(
        _node_enc_body,
        out_shape=[jax.ShapeDtypeStruct((n, enc), F32),
                   jax.ShapeDtypeStruct((n, 2 * enc), F32)],
    )(node, Wn1, row2(bn1), Wn2, row2(bn2))

    be_blk = 2000
    assert e % be_blk == 0
    full = lambda shape: pl.BlockSpec(shape, lambda i: (0,) * len(shape))
    ef = pl.pallas_call(
        _edge_enc_body,
        grid=(e // be_blk,),
        in_specs=[pl.BlockSpec((be_blk, 1), lambda i: (i, 0)),
                  full(We1.shape), full((1, We1.shape[1])),
                  full(We2.shape), full((1, We2.shape[1]))],
        out_specs=pl.BlockSpec((be_blk, enc), lambda i: (i, 0)),
        out_shape=jax.ShapeDtypeStruct((e, enc), F32),
    )(edge, We1, row2(be1), We2, row2(be2))

    if _SC_STAGE >= 1:
        sc_gather = _make_gather(n, e, 2 * enc)
    if _SC_STAGE >= 2:
        sc_scatter = _make_scatter(n, e, hid)
    ncores, _ = _sc_info()

    for i in range(nlayers):
        if _SC_STAGE >= 1:
            hf, ht = sc_gather(hp, frm, to)
        else:
            hf = hp[frm]
            ht = hp[to]
        msg = pl.pallas_call(
            functools.partial(_msg_body, hid),
            grid=(e // be_blk,),
            in_specs=[pl.BlockSpec((be_blk, 2 * enc), lambda i: (i, 0)),
                      pl.BlockSpec((be_blk, 2 * enc), lambda i: (i, 0)),
                      pl.BlockSpec((be_blk, enc), lambda i: (i, 0)),
                      full(Wm1[0].shape), full((1, hid)),
                      full(Wm2[0].shape), full((1, hid))],
            out_specs=pl.BlockSpec((be_blk, 2 * hid), lambda i: (i, 0)),
            out_shape=jax.ShapeDtypeStruct((e, 2 * hid), F32),
        )(hf, ht, ef, Wm1[i], row2(bm1[i]), Wm2[i], row2(bm2[i]))
        if _SC_STAGE >= 2:
            s = sc_scatter(msg, to)
        else:
            s_flat = jax.ops.segment_sum(msg, to, num_segments=n)
            s = jnp.stack([s_flat, jnp.zeros_like(s_flat)])
        if i + 1 < nlayers:
            h, hp = pl.pallas_call(
                functools.partial(_update_body, ncores, n, hid),
                out_shape=[jax.ShapeDtypeStruct((n, enc), F32),
                           jax.ShapeDtypeStruct((n, 2 * enc), F32)],
            )(h, s, Wu1[i], row2(bu1[i]), Wu2[i], row2(bu2[i]))
        else:
            out = pl.pallas_call(
                functools.partial(_final_body, ncores, n, hid, gr),
                out_shape=jax.ShapeDtypeStruct((G, gr), F32),
            )(h, s, Wu1[i], row2(bu1[i]), Wu2[i], row2(bu2[i]),
              Wa1, row2(ba1), Wa2, row2(ba2), graph_idx.reshape(n, 1))
    return out
